# baseline (speedup 1.0000x reference)
import functools

import jax
import jax.numpy as jnp
from jax import lax
from jax.experimental import pallas as pl
from jax.experimental.pallas import tpu as pltpu
from jax.experimental.pallas import tpu_sc as plsc

NC = 2
NS = 16
NW = NC * NS
LANES = 16
CH = 128
DEGW = 16
DH = 64


def _sc_mesh():
    return plsc.VectorSubcoreMesh(core_axis_name="c", subcore_axis_name="s")


def _make_sc_degrees(R, nchunks):
    rows_per_tile = R // NS
    n_init = rows_per_tile // CH

    @functools.partial(
        pl.kernel,
        out_type=(
            jax.ShapeDtypeStruct((NC, R, DEGW), jnp.float32),
            jax.ShapeDtypeStruct((NC, R, DEGW), jnp.float32),
        ),
        mesh=_sc_mesh(),
        scratch_types=[
            pltpu.VMEM((nchunks, CH), jnp.int32),
            pltpu.VMEM((nchunks, CH), jnp.int32),
            pltpu.VMEM((CH, DEGW), jnp.float32),
            pltpu.VMEM_SHARED((R, DEGW), jnp.float32),
            pltpu.VMEM_SHARED((R, DEGW), jnp.float32),
        ],
        compiler_params=pltpu.CompilerParams(use_tc_tiling_on_sc=False),
    )
    def body(src_hbm, dst_hbm, osrc_hbm, odst_hbm,
             src_v, dst_v, ones_v, hsrc_sh, hdst_sh):
        c = lax.axis_index("c")
        s = lax.axis_index("s")
        wid = s * NC + c
        pltpu.sync_copy(src_hbm.at[wid], src_v)
        pltpu.sync_copy(dst_hbm.at[wid], dst_v)

        zero = jnp.zeros((LANES,), jnp.float32)
        one = jnp.ones((LANES,), jnp.float32)

        @pl.loop(0, CH)
        def _z(i):
            ones_v[i, :] = zero

        base = s * rows_per_tile
        for k in range(n_init):
            pltpu.sync_copy(ones_v, hsrc_sh.at[pl.ds(base + k * CH, CH)])
            pltpu.sync_copy(ones_v, hdst_sh.at[pl.ds(base + k * CH, CH)])

        @pl.loop(0, CH)
        def _o(i):
            ones_v[i, :] = one

        plsc.subcore_barrier()

        @pl.loop(0, nchunks)
        def _scat(j):
            pltpu.sync_copy(ones_v, hsrc_sh.at[src_v.at[j]], add=True)
            pltpu.sync_copy(ones_v, hdst_sh.at[dst_v.at[j]], add=True)

        plsc.subcore_barrier()
        sl = pl.ds(base, rows_per_tile)
        pltpu.sync_copy(hsrc_sh.at[sl], osrc_hbm.at[c, sl])
        pltpu.sync_copy(hdst_sh.at[sl], odst_hbm.at[c, sl])

    return body


def _make_sc_layer(R, nchunks):
    rows_per_tile = R // NS
    n_init = rows_per_tile // CH

    @functools.partial(
        pl.kernel,
        out_type=(
            jax.ShapeDtypeStruct((NC, R, DH), jnp.float32),
            jax.ShapeDtypeStruct((NC, R, DH), jnp.float32),
        ),
        mesh=_sc_mesh(),
        scratch_types=[
            pltpu.VMEM((nchunks, CH), jnp.int32),
            pltpu.VMEM((nchunks, CH), jnp.int32),
            pltpu.VMEM((8, CH, DH), jnp.float32),
            pltpu.VMEM_SHARED((R, DH), jnp.float32),
            pltpu.SemaphoreType.DMA,
            pltpu.SemaphoreType.DMA,
        ],
        compiler_params=pltpu.CompilerParams(use_tc_tiling_on_sc=False),
    )
    def body(z0_hbm, z1_hbm, src_hbm, dst_hbm, out0_hbm, out1_hbm,
             src_v, dst_v, rows_v, agg_sh, sem, sem_s):
        c = lax.axis_index("c")
        s = lax.axis_index("s")
        wid = s * NC + c
        pltpu.sync_copy(src_hbm.at[wid], src_v)
        pltpu.sync_copy(dst_hbm.at[wid], dst_v)

        zero = jnp.zeros((LANES,), jnp.float32)
        base = s * rows_per_tile
        sl = pl.ds(base, rows_per_tile)

        for z_hbm, out_hbm in ((z0_hbm, out0_hbm), (z1_hbm, out1_hbm)):
            @pl.loop(0, CH)
            def _z(i):
                for q in range(DH // LANES):
                    rows_v[0, i, pl.ds(q * LANES, LANES)] = zero

            for k in range(n_init):
                pltpu.sync_copy(rows_v.at[0], agg_sh.at[pl.ds(base + k * CH, CH)])
            plsc.subcore_barrier()

            pltpu.async_copy(z_hbm.at[src_v.at[0]], rows_v.at[0], sem)

            @pl.loop(0, nchunks, step=8)
            def _main(j):
                for b in range(8):
                    jb = j + b
                    pltpu.make_async_copy(
                        z_hbm.at[src_v.at[0]], rows_v.at[b], sem).wait()
                    pltpu.async_copy(rows_v.at[b], agg_sh.at[dst_v.at[jb]],
                                     sem_s, add=True)

                    @pl.when(jb + 1 < nchunks)
                    def _prefetch():
                        @pl.when(jb >= 7)
                        def _drain():
                            pltpu.make_async_copy(
                                z_hbm.at[src_v.at[0]],
                                rows_v.at[(b + 1) % 8], sem_s).wait()

                        pltpu.async_copy(
                            z_hbm.at[src_v.at[jb + 1]],
                            rows_v.at[(b + 1) % 8], sem)

            for _ in range(8):
                pltpu.make_async_copy(
                    z_hbm.at[src_v.at[0]], rows_v.at[0], sem_s).wait()
            plsc.subcore_barrier()
            pltpu.sync_copy(agg_sh.at[sl], out_hbm.at[c, sl])

    return body


def _tc_pre(dsrc_p, ddst_p, feat, W1):
    R, D = feat.shape

    def body(dsp, ddp, x, w, z0_out, z1_out, ns_out, nd_out):
        ds_ = dsp[0, :, 0:1] + dsp[1, :, 0:1]
        dd_ = ddp[0, :, 0:1] + ddp[1, :, 0:1]
        ns = lax.rsqrt(jnp.maximum(ds_, 1.0))
        nd = lax.rsqrt(jnp.maximum(dd_, 1.0))
        ns_out[...] = ns
        nd_out[...] = nd
        z = jnp.dot(x[...] * ns, w[...], preferred_element_type=jnp.float32)
        z0_out[...] = z[:, :DH]
        z1_out[...] = z[:, DH:]

    return pl.pallas_call(
        body,
        out_shape=(
            jax.ShapeDtypeStruct((R, DH), jnp.float32),
            jax.ShapeDtypeStruct((R, DH), jnp.float32),
            jax.ShapeDtypeStruct((R, 1), jnp.float32),
            jax.ShapeDtypeStruct((R, 1), jnp.float32),
        ),
    )(dsrc_p, ddst_p, feat, W1)


def _tc_mid(p0, p1, ns, nd, b1, W2):
    _, R, _ = p0.shape

    def body(p0_ref, p1_ref, ns_ref, nd_ref, b_ref, w_ref, z0_out, z1_out):
        h = jnp.concatenate(
            [p0_ref[0] + p0_ref[1], p1_ref[0] + p1_ref[1]], axis=1)
        h = jnp.maximum(h * nd_ref[...] + b_ref[...], 0.0)
        z = jnp.dot(h * ns_ref[...], w_ref[...],
                    preferred_element_type=jnp.float32)
        z0_out[...] = z[:, :DH]
        z1_out[...] = z[:, DH:]

    return pl.pallas_call(
        body,
        out_shape=(
            jax.ShapeDtypeStruct((R, DH), jnp.float32),
            jax.ShapeDtypeStruct((R, DH), jnp.float32),
        ),
    )(p0, p1, ns, nd, b1, W2)


def _tc_post(p0, p1, nd, b2):
    _, R, _ = p0.shape

    def body(p0_ref, p1_ref, nd_ref, b_ref, out):
        h = jnp.concatenate(
            [p0_ref[0] + p0_ref[1], p1_ref[0] + p1_ref[1]], axis=1)
        out[...] = h * nd_ref[...] + b_ref[...]

    return pl.pallas_call(
        body,
        out_shape=jax.ShapeDtypeStruct((R, 2 * DH), jnp.float32),
    )(p0, p1, nd, b2)


def kernel(features, edge_index, W1, b1, W2, b2):
    N, D = features.shape
    E = edge_index.shape[1]

    R = pl.cdiv(N + 1, NS * CH) * (NS * CH)
    epw = pl.cdiv(E, NW * 4 * CH) * (4 * CH)
    nchunks = epw // CH
    pad = NW * epw - E

    src = edge_index[0]
    dst = edge_index[1]
    trash = (N + jnp.arange(pad, dtype=jnp.int32) % (R - N)).astype(jnp.int32)
    srcp = jnp.concatenate([src, trash]).reshape(NW, nchunks, CH)
    dstp = jnp.concatenate([dst, trash]).reshape(NW, nchunks, CH)

    feat = jnp.pad(features, ((0, R - N), (0, 0)))

    sc_deg = _make_sc_degrees(R, nchunks)
    sc_layer = _make_sc_layer(R, nchunks)

    dsrc_p, ddst_p = sc_deg(srcp, dstp)
    z1a, z1b, ns, nd = _tc_pre(dsrc_p, ddst_p, feat, W1)
    p1a, p1b = sc_layer(z1a, z1b, srcp, dstp)
    z2a, z2b = _tc_mid(p1a, p1b, ns, nd, b1.reshape(1, D), W2)
    p2a, p2b = sc_layer(z2a, z2b, srcp, dstp)
    out = _tc_post(p2a, p2b, nd, b2.reshape(1, D))
    return out[:N]


# speedup vs baseline: 1.2473x; 1.2473x over previous
import functools

import jax
import jax.numpy as jnp
from jax import lax
from jax.experimental import pallas as pl
from jax.experimental.pallas import tpu as pltpu
from jax.experimental.pallas import tpu_sc as plsc

NC = 2
NS = 16
NW = NC * NS
LANES = 16
CH = 128
DEGW = 16
DH = 64


def _sc_mesh():
    return plsc.VectorSubcoreMesh(core_axis_name="c", subcore_axis_name="s")


def _make_sc_degrees(R, nchunks):
    rows_per_tile = R // NS
    n_init = rows_per_tile // CH

    @functools.partial(
        pl.kernel,
        out_type=(
            jax.ShapeDtypeStruct((NC, R, DEGW), jnp.float32),
            jax.ShapeDtypeStruct((NC, R, DEGW), jnp.float32),
        ),
        mesh=_sc_mesh(),
        scratch_types=[
            pltpu.VMEM((nchunks, CH), jnp.int32),
            pltpu.VMEM((nchunks, CH), jnp.int32),
            pltpu.VMEM((CH, DEGW), jnp.float32),
            pltpu.VMEM_SHARED((R, DEGW), jnp.float32),
            pltpu.VMEM_SHARED((R, DEGW), jnp.float32),
        ],
        compiler_params=pltpu.CompilerParams(use_tc_tiling_on_sc=False),
    )
    def body(src_hbm, dst_hbm, osrc_hbm, odst_hbm,
             src_v, dst_v, ones_v, hsrc_sh, hdst_sh):
        c = lax.axis_index("c")
        s = lax.axis_index("s")
        wid = s * NC + c
        pltpu.sync_copy(src_hbm.at[wid], src_v)
        pltpu.sync_copy(dst_hbm.at[wid], dst_v)

        zero = jnp.zeros((LANES,), jnp.float32)
        one = jnp.ones((LANES,), jnp.float32)

        @pl.loop(0, CH)
        def _z(i):
            ones_v[i, :] = zero

        base = s * rows_per_tile
        for k in range(n_init):
            pltpu.sync_copy(ones_v, hsrc_sh.at[pl.ds(base + k * CH, CH)])
            pltpu.sync_copy(ones_v, hdst_sh.at[pl.ds(base + k * CH, CH)])

        @pl.loop(0, CH)
        def _o(i):
            ones_v[i, :] = one

        plsc.subcore_barrier()

        @pl.loop(0, nchunks)
        def _scat(j):
            pltpu.sync_copy(ones_v, hsrc_sh.at[src_v.at[j]], add=True)
            pltpu.sync_copy(ones_v, hdst_sh.at[dst_v.at[j]], add=True)

        plsc.subcore_barrier()
        sl = pl.ds(base, rows_per_tile)
        pltpu.sync_copy(hsrc_sh.at[sl], osrc_hbm.at[c, sl])
        pltpu.sync_copy(hdst_sh.at[sl], odst_hbm.at[c, sl])

    return body


def _make_sc_layer(R, nchunks):
    rows_per_tile = R // NS
    n_init = rows_per_tile // CH

    @functools.partial(
        pl.kernel,
        out_type=(
            jax.ShapeDtypeStruct((NC, R, DH), jnp.float32),
            jax.ShapeDtypeStruct((NC, R, DH), jnp.float32),
        ),
        mesh=_sc_mesh(),
        scratch_types=[
            pltpu.VMEM((nchunks, CH), jnp.int32),
            pltpu.VMEM((nchunks, CH), jnp.int32),
            pltpu.VMEM((4, CH, DH), jnp.float32),
            pltpu.VMEM_SHARED((R, DH), jnp.float32),
            pltpu.SemaphoreType.DMA,
            pltpu.SemaphoreType.DMA,
            pltpu.SemaphoreType.DMA,
            pltpu.SemaphoreType.DMA,
        ],
        compiler_params=pltpu.CompilerParams(use_tc_tiling_on_sc=False),
    )
    def body(z0_hbm, z1_hbm, src_hbm, dst_hbm, out0_hbm, out1_hbm,
             src_v, dst_v, rows_v, agg_sh, sem_g0, sem_g1, sem_s0, sem_s1):
        c = lax.axis_index("c")
        s = lax.axis_index("s")
        wid = s * NC + c
        pltpu.sync_copy(src_hbm.at[wid], src_v)
        pltpu.sync_copy(dst_hbm.at[wid], dst_v)

        zero = jnp.zeros((LANES,), jnp.float32)
        base = s * rows_per_tile
        sl = pl.ds(base, rows_per_tile)

        for z_hbm, out_hbm in ((z0_hbm, out0_hbm), (z1_hbm, out1_hbm)):
            @pl.loop(0, CH)
            def _z(i):
                for q in range(DH // LANES):
                    rows_v[0, i, pl.ds(q * LANES, LANES)] = zero

            for k in range(n_init):
                pltpu.sync_copy(rows_v.at[0], agg_sh.at[pl.ds(base + k * CH, CH)])
            plsc.subcore_barrier()

            sems_g = (sem_g0, sem_g1)
            sems_s = (sem_s0, sem_s1)
            pltpu.async_copy(z_hbm.at[src_v.at[0]], rows_v.at[0], sem_g0)
            pltpu.async_copy(z_hbm.at[src_v.at[1]], rows_v.at[1], sem_g1)

            @pl.loop(0, nchunks, step=4)
            def _main(j):
                for b in range(4):
                    jb = j + b
                    par = b % 2
                    pltpu.make_async_copy(
                        z_hbm.at[src_v.at[0]], rows_v.at[b], sems_g[par]).wait()

                    @pl.when(jb >= 2)
                    def _drain():
                        pltpu.make_async_copy(
                            z_hbm.at[src_v.at[0]],
                            rows_v.at[(b + 2) % 4], sems_s[par]).wait()

                    pltpu.async_copy(rows_v.at[b], agg_sh.at[dst_v.at[jb]],
                                     sems_s[par], add=True)

                    @pl.when(jb + 2 < nchunks)
                    def _prefetch():
                        pltpu.async_copy(
                            z_hbm.at[src_v.at[jb + 2]],
                            rows_v.at[(b + 2) % 4], sems_g[par])

            for par in range(2):
                pltpu.make_async_copy(
                    z_hbm.at[src_v.at[0]], rows_v.at[0], sems_s[par]).wait()
            plsc.subcore_barrier()
            pltpu.sync_copy(agg_sh.at[sl], out_hbm.at[c, sl])

    return body


def _tc_pre(dsrc_p, ddst_p, feat, W1):
    R, D = feat.shape

    def body(dsp, ddp, x, w, z0_out, z1_out, ns_out, nd_out):
        ds_ = dsp[0, :, 0:1] + dsp[1, :, 0:1]
        dd_ = ddp[0, :, 0:1] + ddp[1, :, 0:1]
        ns = lax.rsqrt(jnp.maximum(ds_, 1.0))
        nd = lax.rsqrt(jnp.maximum(dd_, 1.0))
        ns_out[...] = ns
        nd_out[...] = nd
        z = jnp.dot(x[...] * ns, w[...], preferred_element_type=jnp.float32)
        z0_out[...] = z[:, :DH]
        z1_out[...] = z[:, DH:]

    return pl.pallas_call(
        body,
        out_shape=(
            jax.ShapeDtypeStruct((R, DH), jnp.float32),
            jax.ShapeDtypeStruct((R, DH), jnp.float32),
            jax.ShapeDtypeStruct((R, 1), jnp.float32),
            jax.ShapeDtypeStruct((R, 1), jnp.float32),
        ),
    )(dsrc_p, ddst_p, feat, W1)


def _tc_mid(p0, p1, ns, nd, b1, W2):
    _, R, _ = p0.shape

    def body(p0_ref, p1_ref, ns_ref, nd_ref, b_ref, w_ref, z0_out, z1_out):
        h = jnp.concatenate(
            [p0_ref[0] + p0_ref[1], p1_ref[0] + p1_ref[1]], axis=1)
        h = jnp.maximum(h * nd_ref[...] + b_ref[...], 0.0)
        z = jnp.dot(h * ns_ref[...], w_ref[...],
                    preferred_element_type=jnp.float32)
        z0_out[...] = z[:, :DH]
        z1_out[...] = z[:, DH:]

    return pl.pallas_call(
        body,
        out_shape=(
            jax.ShapeDtypeStruct((R, DH), jnp.float32),
            jax.ShapeDtypeStruct((R, DH), jnp.float32),
        ),
    )(p0, p1, ns, nd, b1, W2)


def _tc_post(p0, p1, nd, b2):
    _, R, _ = p0.shape

    def body(p0_ref, p1_ref, nd_ref, b_ref, out):
        h = jnp.concatenate(
            [p0_ref[0] + p0_ref[1], p1_ref[0] + p1_ref[1]], axis=1)
        out[...] = h * nd_ref[...] + b_ref[...]

    return pl.pallas_call(
        body,
        out_shape=jax.ShapeDtypeStruct((R, 2 * DH), jnp.float32),
    )(p0, p1, nd, b2)


def kernel(features, edge_index, W1, b1, W2, b2):
    N, D = features.shape
    E = edge_index.shape[1]

    R = pl.cdiv(N + 1, NS * CH) * (NS * CH)
    epw = pl.cdiv(E, NW * 4 * CH) * (4 * CH)
    nchunks = epw // CH
    pad = NW * epw - E

    src = edge_index[0]
    dst = edge_index[1]
    trash = (N + jnp.arange(pad, dtype=jnp.int32) % (R - N)).astype(jnp.int32)
    srcp = jnp.concatenate([src, trash]).reshape(NW, nchunks, CH)
    dstp = jnp.concatenate([dst, trash]).reshape(NW, nchunks, CH)

    feat = jnp.pad(features, ((0, R - N), (0, 0)))

    sc_deg = _make_sc_degrees(R, nchunks)
    sc_layer = _make_sc_layer(R, nchunks)

    dsrc_p, ddst_p = sc_deg(srcp, dstp)
    z1a, z1b, ns, nd = _tc_pre(dsrc_p, ddst_p, feat, W1)
    p1a, p1b = sc_layer(z1a, z1b, srcp, dstp)
    z2a, z2b = _tc_mid(p1a, p1b, ns, nd, b1.reshape(1, D), W2)
    p2a, p2b = sc_layer(z2a, z2b, srcp, dstp)
    out = _tc_post(p2a, p2b, nd, b2.reshape(1, D))
    return out[:N]


# speedup vs baseline: 1.3921x; 1.1160x over previous
import functools

import jax
import jax.numpy as jnp
from jax import lax
from jax.experimental import pallas as pl
from jax.experimental.pallas import tpu as pltpu
from jax.experimental.pallas import tpu_sc as plsc

NC = 2
NS = 16
LANES = 16
CH = 128
DEGW = 16
DH = 64


def _sc_mesh():
    return plsc.VectorSubcoreMesh(core_axis_name="c", subcore_axis_name="s")


def _make_sc_degrees(R, nchunks):
    rows_per_tile = R // NS
    n_init = rows_per_tile // CH

    @functools.partial(
        pl.kernel,
        out_type=jax.ShapeDtypeStruct((NC, R, DEGW), jnp.float32),
        mesh=_sc_mesh(),
        scratch_types=[
            pltpu.VMEM((nchunks, CH), jnp.int32),
            pltpu.VMEM((CH, DEGW), jnp.float32),
            pltpu.VMEM_SHARED((R, DEGW), jnp.float32),
            pltpu.SemaphoreType.DMA,
        ],
        compiler_params=pltpu.CompilerParams(use_tc_tiling_on_sc=False),
    )
    def body(edges_hbm, odeg_hbm, idx_v, ones_v, hist_sh, sem):
        c = lax.axis_index("c")
        s = lax.axis_index("s")
        pltpu.sync_copy(edges_hbm.at[c, s], idx_v)

        zero = jnp.zeros((LANES,), jnp.float32)
        one = jnp.ones((LANES,), jnp.float32)

        @pl.loop(0, CH)
        def _z(i):
            ones_v[i, :] = zero

        base = s * rows_per_tile
        for k in range(n_init):
            pltpu.sync_copy(ones_v, hist_sh.at[pl.ds(base + k * CH, CH)])

        @pl.loop(0, CH)
        def _o(i):
            ones_v[i, :] = one

        plsc.subcore_barrier()

        @pl.loop(0, nchunks)
        def _scat(j):
            pltpu.async_copy(ones_v, hist_sh.at[idx_v.at[j]], sem, add=True)

            @pl.when(j >= 8)
            def _drain():
                pltpu.make_async_copy(odeg_hbm.at[0, pl.ds(0, CH)], ones_v,
                                      sem).wait()

        for _ in range(8):
            pltpu.make_async_copy(odeg_hbm.at[0, pl.ds(0, CH)], ones_v,
                                  sem).wait()

        plsc.subcore_barrier()
        sl = pl.ds(base, rows_per_tile)
        pltpu.sync_copy(hist_sh.at[sl], odeg_hbm.at[c, sl])

    return body


def _make_sc_layer(R, nchunks):
    rows_per_tile = R // NS
    n_init = rows_per_tile // CH

    @functools.partial(
        pl.kernel,
        out_type=jax.ShapeDtypeStruct((NC, R, DH), jnp.float32),
        mesh=_sc_mesh(),
        scratch_types=[
            pltpu.VMEM((nchunks, CH), jnp.int32),
            pltpu.VMEM((nchunks, CH), jnp.int32),
            pltpu.VMEM((4, CH, DH), jnp.float32),
            pltpu.VMEM_SHARED((R, DH), jnp.float32),
            pltpu.SemaphoreType.DMA,
            pltpu.SemaphoreType.DMA,
            pltpu.SemaphoreType.DMA,
            pltpu.SemaphoreType.DMA,
        ],
        compiler_params=pltpu.CompilerParams(use_tc_tiling_on_sc=False),
    )
    def body(z_hbm2, src_hbm, dst_hbm, out_hbm2,
             src_v, dst_v, rows_v, agg_sh, sem_g0, sem_g1, sem_s0, sem_s1):
        c = lax.axis_index("c")
        s = lax.axis_index("s")
        pltpu.sync_copy(src_hbm.at[s], src_v)
        pltpu.sync_copy(dst_hbm.at[s], dst_v)
        z_hbm = z_hbm2.at[c]

        zero = jnp.zeros((LANES,), jnp.float32)
        base = s * rows_per_tile
        sl = pl.ds(base, rows_per_tile)

        @pl.loop(0, CH)
        def _z(i):
            for q in range(DH // LANES):
                rows_v[0, i, pl.ds(q * LANES, LANES)] = zero

        for k in range(n_init):
            pltpu.sync_copy(rows_v.at[0], agg_sh.at[pl.ds(base + k * CH, CH)])
        plsc.subcore_barrier()

        sems_g = (sem_g0, sem_g1)
        sems_s = (sem_s0, sem_s1)
        pltpu.async_copy(z_hbm.at[src_v.at[0]], rows_v.at[0], sem_g0)
        pltpu.async_copy(z_hbm.at[src_v.at[1]], rows_v.at[1], sem_g1)

        @pl.loop(0, nchunks, step=4)
        def _main(j):
            for b in range(4):
                jb = j + b
                par = b % 2
                pltpu.make_async_copy(
                    z_hbm.at[src_v.at[0]], rows_v.at[b], sems_g[par]).wait()

                @pl.when(jb >= 2)
                def _drain():
                    pltpu.make_async_copy(
                        z_hbm.at[src_v.at[0]],
                        rows_v.at[(b + 2) % 4], sems_s[par]).wait()

                pltpu.async_copy(rows_v.at[b], agg_sh.at[dst_v.at[jb]],
                                 sems_s[par], add=True)

                @pl.when(jb + 2 < nchunks)
                def _prefetch():
                    pltpu.async_copy(
                        z_hbm.at[src_v.at[jb + 2]],
                        rows_v.at[(b + 2) % 4], sems_g[par])

        for par in range(2):
            pltpu.make_async_copy(
                z_hbm.at[src_v.at[0]], rows_v.at[0], sems_s[par]).wait()
        plsc.subcore_barrier()
        pltpu.sync_copy(agg_sh.at[sl], out_hbm2.at[c, sl])

    return body


def _tc_pre(deg, feat, W1):
    R, D = feat.shape

    def body(dg, x, w, z_out, ns_out, nd_out):
        ns = lax.rsqrt(jnp.maximum(dg[0, :, 0:1], 1.0))
        nd = lax.rsqrt(jnp.maximum(dg[1, :, 0:1], 1.0))
        ns_out[...] = ns
        nd_out[...] = nd
        z = jnp.dot(x[...] * ns, w[...], preferred_element_type=jnp.float32)
        z_out[0] = z[:, :DH]
        z_out[1] = z[:, DH:]

    return pl.pallas_call(
        body,
        out_shape=(
            jax.ShapeDtypeStruct((NC, R, DH), jnp.float32),
            jax.ShapeDtypeStruct((R, 1), jnp.float32),
            jax.ShapeDtypeStruct((R, 1), jnp.float32),
        ),
    )(deg, feat, W1)


def _tc_mid(p, ns, nd, b1, W2):
    _, R, _ = p.shape

    def body(p_ref, ns_ref, nd_ref, b_ref, w_ref, z_out):
        h = jnp.concatenate([p_ref[0], p_ref[1]], axis=1)
        h = jnp.maximum(h * nd_ref[...] + b_ref[...], 0.0)
        z = jnp.dot(h * ns_ref[...], w_ref[...],
                    preferred_element_type=jnp.float32)
        z_out[0] = z[:, :DH]
        z_out[1] = z[:, DH:]

    return pl.pallas_call(
        body,
        out_shape=jax.ShapeDtypeStruct((NC, R, DH), jnp.float32),
    )(p, ns, nd, b1, W2)


def _tc_post(p, nd, b2):
    _, R, _ = p.shape

    def body(p_ref, nd_ref, b_ref, out):
        h = jnp.concatenate([p_ref[0], p_ref[1]], axis=1)
        out[...] = h * nd_ref[...] + b_ref[...]

    return pl.pallas_call(
        body,
        out_shape=jax.ShapeDtypeStruct((R, 2 * DH), jnp.float32),
    )(p, nd, b2)


def kernel(features, edge_index, W1, b1, W2, b2):
    N, D = features.shape
    E = edge_index.shape[1]

    R = pl.cdiv(N + 1, NS * CH) * (NS * CH)
    ept = pl.cdiv(E, NS * 4 * CH) * (4 * CH)
    nchunks = ept // CH
    pad = NS * ept - E

    src = edge_index[0]
    dst = edge_index[1]
    trash = (N + jnp.arange(pad, dtype=jnp.int32) % (R - N)).astype(jnp.int32)
    srcp = jnp.concatenate([src, trash]).reshape(NS, nchunks, CH)
    dstp = jnp.concatenate([dst, trash]).reshape(NS, nchunks, CH)
    edges = jnp.stack([srcp, dstp])

    feat = jnp.pad(features, ((0, R - N), (0, 0)))

    sc_deg = _make_sc_degrees(R, nchunks)
    sc_layer = _make_sc_layer(R, nchunks)

    deg = sc_deg(edges)
    z1, ns, nd = _tc_pre(deg, feat, W1)
    p1 = sc_layer(z1, srcp, dstp)
    z2 = _tc_mid(p1, ns, nd, b1.reshape(1, D), W2)
    p2 = sc_layer(z2, srcp, dstp)
    out = _tc_post(p2, nd, b2.reshape(1, D))
    return out[:N]


# speedup vs baseline: 1.4937x; 1.0730x over previous
import functools

import jax
import jax.numpy as jnp
from jax import lax
from jax.experimental import pallas as pl
from jax.experimental.pallas import tpu as pltpu
from jax.experimental.pallas import tpu_sc as plsc

NC = 2
NS = 16
LANES = 16
CH = 128
DEGW = 16
DH = 64


def _sc_mesh():
    return plsc.VectorSubcoreMesh(core_axis_name="c", subcore_axis_name="s")


def _make_sc_degrees(R, nchunks):
    rows_per_tile = R // NS
    n_init = rows_per_tile // CH

    @functools.partial(
        pl.kernel,
        out_type=jax.ShapeDtypeStruct((NC, R, DEGW), jnp.float32),
        mesh=_sc_mesh(),
        scratch_types=[
            pltpu.VMEM((nchunks, CH), jnp.int32),
            pltpu.VMEM((CH, DEGW), jnp.float32),
            pltpu.VMEM_SHARED((R, DEGW), jnp.float32),
            pltpu.SemaphoreType.DMA,
        ],
        compiler_params=pltpu.CompilerParams(use_tc_tiling_on_sc=False),
    )
    def body(edges_hbm, odeg_hbm, idx_v, ones_v, hist_sh, sem):
        c = lax.axis_index("c")
        s = lax.axis_index("s")
        pltpu.sync_copy(edges_hbm.at[c, s], idx_v)

        zero = jnp.zeros((LANES,), jnp.float32)
        one = jnp.ones((LANES,), jnp.float32)

        @pl.loop(0, CH)
        def _z(i):
            ones_v[i, :] = zero

        base = s * rows_per_tile
        for k in range(n_init):
            pltpu.sync_copy(ones_v, hist_sh.at[pl.ds(base + k * CH, CH)])

        @pl.loop(0, CH)
        def _o(i):
            ones_v[i, :] = one

        plsc.subcore_barrier()

        @pl.loop(0, nchunks)
        def _scat(j):
            pltpu.async_copy(ones_v, hist_sh.at[idx_v.at[j]], sem, add=True)

            @pl.when(j >= 8)
            def _drain():
                pltpu.make_async_copy(odeg_hbm.at[0, pl.ds(0, CH)], ones_v,
                                      sem).wait()

        for _ in range(8):
            pltpu.make_async_copy(odeg_hbm.at[0, pl.ds(0, CH)], ones_v,
                                  sem).wait()

        plsc.subcore_barrier()
        sl = pl.ds(base, rows_per_tile)
        pltpu.sync_copy(hist_sh.at[sl], odeg_hbm.at[c, sl])

    return body


def _make_sc_layer(R, nchunks):
    rows_per_tile = R // NS
    n_init = rows_per_tile // CH

    @functools.partial(
        pl.kernel,
        out_type=jax.ShapeDtypeStruct((NC, R, DH), jnp.float32),
        mesh=_sc_mesh(),
        scratch_types=[
            pltpu.VMEM((nchunks // 2, CH), jnp.int32),
            pltpu.VMEM((nchunks // 2, CH), jnp.int32),
            pltpu.VMEM((8, CH, DH), jnp.float32),
            pltpu.VMEM_SHARED((R, DH), jnp.float32),
            [pltpu.SemaphoreType.DMA] * 4,
            [pltpu.SemaphoreType.DMA] * 4,
        ],
        compiler_params=pltpu.CompilerParams(use_tc_tiling_on_sc=False),
    )
    def body(z_hbm2, src_hbm, dst_hbm, out_hbm2,
             src_v, dst_v, rows_v, agg_sh, sems_g, sems_s):
        c = lax.axis_index("c")
        s = lax.axis_index("s")
        hc = nchunks // 2
        z_hbm = z_hbm2.at[c]

        zero = jnp.zeros((LANES,), jnp.float32)
        base = s * rows_per_tile
        sl = pl.ds(base, rows_per_tile)

        @pl.loop(0, CH)
        def _z(i):
            for q in range(DH // LANES):
                rows_v[0, i, pl.ds(q * LANES, LANES)] = zero

        for k in range(n_init):
            pltpu.sync_copy(rows_v.at[0], agg_sh.at[pl.ds(base + k * CH, CH)])
        plsc.subcore_barrier()

        for half in range(2):
            pltpu.sync_copy(src_hbm.at[s, pl.ds(half * hc, hc)], src_v)
            pltpu.sync_copy(dst_hbm.at[s, pl.ds(half * hc, hc)], dst_v)

            for b in range(4):
                pltpu.async_copy(z_hbm.at[src_v.at[b]], rows_v.at[b],
                                 sems_g[b])

            @pl.loop(0, hc, step=8)
            def _main(j):
                for b in range(8):
                    jb = j + b
                    par = b % 4
                    pltpu.make_async_copy(
                        z_hbm.at[src_v.at[0]], rows_v.at[b],
                        sems_g[par]).wait()

                    @pl.when(jb >= 4)
                    def _drain():
                        pltpu.make_async_copy(
                            z_hbm.at[src_v.at[0]],
                            rows_v.at[(b + 4) % 8], sems_s[par]).wait()

                    pltpu.async_copy(rows_v.at[b], agg_sh.at[dst_v.at[jb]],
                                     sems_s[par], add=True)

                    @pl.when(jb + 4 < hc)
                    def _prefetch():
                        pltpu.async_copy(
                            z_hbm.at[src_v.at[jb + 4]],
                            rows_v.at[(b + 4) % 8], sems_g[par])

            for par in range(4):
                pltpu.make_async_copy(
                    z_hbm.at[src_v.at[0]], rows_v.at[0], sems_s[par]).wait()
        plsc.subcore_barrier()
        pltpu.sync_copy(agg_sh.at[sl], out_hbm2.at[c, sl])

    return body


def _tc_pre(deg, feat, W1):
    R, D = feat.shape

    def body(dg, x, w, z_out, ns_out, nd_out):
        ns = lax.rsqrt(jnp.maximum(dg[0, :, 0:1], 1.0))
        nd = lax.rsqrt(jnp.maximum(dg[1, :, 0:1], 1.0))
        ns_out[...] = ns
        nd_out[...] = nd
        z = jnp.dot(x[...] * ns, w[...], preferred_element_type=jnp.float32)
        z_out[0] = z[:, :DH]
        z_out[1] = z[:, DH:]

    return pl.pallas_call(
        body,
        out_shape=(
            jax.ShapeDtypeStruct((NC, R, DH), jnp.float32),
            jax.ShapeDtypeStruct((R, 1), jnp.float32),
            jax.ShapeDtypeStruct((R, 1), jnp.float32),
        ),
    )(deg, feat, W1)


def _tc_mid(p, ns, nd, b1, W2):
    _, R, _ = p.shape

    def body(p_ref, ns_ref, nd_ref, b_ref, w_ref, z_out):
        h = jnp.concatenate([p_ref[0], p_ref[1]], axis=1)
        h = jnp.maximum(h * nd_ref[...] + b_ref[...], 0.0)
        z = jnp.dot(h * ns_ref[...], w_ref[...],
                    preferred_element_type=jnp.float32)
        z_out[0] = z[:, :DH]
        z_out[1] = z[:, DH:]

    return pl.pallas_call(
        body,
        out_shape=jax.ShapeDtypeStruct((NC, R, DH), jnp.float32),
    )(p, ns, nd, b1, W2)


def _tc_post(p, nd, b2):
    _, R, _ = p.shape

    def body(p_ref, nd_ref, b_ref, out):
        h = jnp.concatenate([p_ref[0], p_ref[1]], axis=1)
        out[...] = h * nd_ref[...] + b_ref[...]

    return pl.pallas_call(
        body,
        out_shape=jax.ShapeDtypeStruct((R, 2 * DH), jnp.float32),
    )(p, nd, b2)


def kernel(features, edge_index, W1, b1, W2, b2):
    N, D = features.shape
    E = edge_index.shape[1]

    R = pl.cdiv(N + 1, NS * CH) * (NS * CH)
    ept = pl.cdiv(E, NS * 16 * CH) * (16 * CH)
    nchunks = ept // CH
    pad = NS * ept - E

    src = edge_index[0]
    dst = edge_index[1]
    trash = (N + jnp.arange(pad, dtype=jnp.int32) % (R - N)).astype(jnp.int32)
    srcp = jnp.concatenate([src, trash]).reshape(NS, nchunks, CH)
    dstp = jnp.concatenate([dst, trash]).reshape(NS, nchunks, CH)
    edges = jnp.stack([srcp, dstp])

    feat = jnp.pad(features, ((0, R - N), (0, 0)))

    sc_deg = _make_sc_degrees(R, nchunks)
    sc_layer = _make_sc_layer(R, nchunks)

    deg = sc_deg(edges)
    z1, ns, nd = _tc_pre(deg, feat, W1)
    p1 = sc_layer(z1, srcp, dstp)
    z2 = _tc_mid(p1, ns, nd, b1.reshape(1, D), W2)
    p2 = sc_layer(z2, srcp, dstp)
    out = _tc_post(p2, nd, b2.reshape(1, D))
    return out[:N]


# speedup vs baseline: 1.5097x; 1.0107x over previous
import functools

import jax
import jax.numpy as jnp
from jax import lax
from jax.experimental import pallas as pl
from jax.experimental.pallas import tpu as pltpu
from jax.experimental.pallas import tpu_sc as plsc

NC = 2
NS = 16
LANES = 16
CH = 128
DEGW = 16
DH = 64


def _sc_mesh():
    return plsc.VectorSubcoreMesh(core_axis_name="c", subcore_axis_name="s")


def _make_sc_degrees(R, nchunks):
    rows_per_tile = R // NS
    n_init = rows_per_tile // CH

    @functools.partial(
        pl.kernel,
        out_type=jax.ShapeDtypeStruct((NC, R, DEGW), jnp.float32),
        mesh=_sc_mesh(),
        scratch_types=[
            pltpu.VMEM((nchunks, CH), jnp.int32),
            pltpu.VMEM((CH, DEGW), jnp.float32),
            pltpu.VMEM_SHARED((R, DEGW), jnp.float32),
            pltpu.SemaphoreType.DMA,
        ],
        compiler_params=pltpu.CompilerParams(use_tc_tiling_on_sc=False),
    )
    def body(edges_hbm, odeg_hbm, idx_v, ones_v, hist_sh, sem):
        c = lax.axis_index("c")
        s = lax.axis_index("s")
        pltpu.sync_copy(edges_hbm.at[c, s], idx_v)

        zero = jnp.zeros((LANES,), jnp.float32)
        one = jnp.ones((LANES,), jnp.float32)

        @pl.loop(0, CH)
        def _z(i):
            ones_v[i, :] = zero

        base = s * rows_per_tile
        for k in range(n_init):
            pltpu.sync_copy(ones_v, hist_sh.at[pl.ds(base + k * CH, CH)])

        @pl.loop(0, CH)
        def _o(i):
            ones_v[i, :] = one

        plsc.subcore_barrier()

        @pl.loop(0, nchunks)
        def _scat(j):
            pltpu.async_copy(ones_v, hist_sh.at[idx_v.at[j]], sem, add=True)

            @pl.when(j >= 8)
            def _drain():
                pltpu.make_async_copy(odeg_hbm.at[0, pl.ds(0, CH)], ones_v,
                                      sem).wait()

        for _ in range(8):
            pltpu.make_async_copy(odeg_hbm.at[0, pl.ds(0, CH)], ones_v,
                                  sem).wait()

        plsc.subcore_barrier()
        sl = pl.ds(base, rows_per_tile)
        pltpu.sync_copy(hist_sh.at[sl], odeg_hbm.at[c, sl])

    return body


def _make_sc_layer(R, nchunks):
    rows_per_tile = R // NS
    n_init = rows_per_tile // CH

    @functools.partial(
        pl.kernel,
        out_type=jax.ShapeDtypeStruct((NC, R, DH), jnp.float32),
        mesh=_sc_mesh(),
        scratch_types=[
            pltpu.VMEM((nchunks // 2, CH), jnp.int32),
            pltpu.VMEM((nchunks // 2, CH), jnp.int32),
            pltpu.VMEM((8, CH, DH), jnp.float32),
            pltpu.VMEM_SHARED((R, DH), jnp.float32),
            [pltpu.SemaphoreType.DMA] * 4,
            [pltpu.SemaphoreType.DMA] * 4,
        ],
        compiler_params=pltpu.CompilerParams(use_tc_tiling_on_sc=False),
    )
    def body(z_hbm2, src_hbm, dst_hbm, out_hbm2,
             src_v, dst_v, rows_v, agg_sh, sems_g, sems_s):
        c = lax.axis_index("c")
        s = lax.axis_index("s")
        hc = nchunks // 2
        z_hbm = z_hbm2.at[c]

        zero = jnp.zeros((LANES,), jnp.float32)
        base = s * rows_per_tile
        sl = pl.ds(base, rows_per_tile)

        @pl.loop(0, CH)
        def _z(i):
            for q in range(DH // LANES):
                rows_v[0, i, pl.ds(q * LANES, LANES)] = zero

        for k in range(n_init):
            pltpu.sync_copy(rows_v.at[0], agg_sh.at[pl.ds(base + k * CH, CH)])
        plsc.subcore_barrier()

        for half in range(2):
            pltpu.sync_copy(src_hbm.at[s, pl.ds(half * hc, hc)], src_v)
            pltpu.sync_copy(dst_hbm.at[s, pl.ds(half * hc, hc)], dst_v)

            for b in range(4):
                pltpu.async_copy(z_hbm.at[src_v.at[b]], rows_v.at[b],
                                 sems_g[b])

            @pl.loop(0, hc, step=8)
            def _main(j):
                for b in range(8):
                    jb = j + b
                    par = b % 4
                    pltpu.make_async_copy(
                        z_hbm.at[src_v.at[0]], rows_v.at[b],
                        sems_g[par]).wait()

                    @pl.when(jb >= 4)
                    def _drain():
                        pltpu.make_async_copy(
                            z_hbm.at[src_v.at[0]],
                            rows_v.at[(b + 4) % 8], sems_s[par]).wait()

                    pltpu.async_copy(rows_v.at[b], agg_sh.at[dst_v.at[jb]],
                                     sems_s[par], add=True)

                    @pl.when(jb + 4 < hc)
                    def _prefetch():
                        pltpu.async_copy(
                            z_hbm.at[src_v.at[jb + 4]],
                            rows_v.at[(b + 4) % 8], sems_g[par])

            for par in range(4):
                pltpu.make_async_copy(
                    z_hbm.at[src_v.at[0]], rows_v.at[0], sems_s[par]).wait()
        plsc.subcore_barrier()
        pltpu.sync_copy(agg_sh.at[sl], out_hbm2.at[c, sl])

    return body


def _tc_mm(feat, W1):
    R, D = feat.shape

    def body(x, w, zr_out):
        zr_out[...] = jnp.dot(x[...], w[...],
                              preferred_element_type=jnp.float32)

    return pl.pallas_call(
        body,
        out_shape=jax.ShapeDtypeStruct((R, D), jnp.float32),
    )(feat, W1)


def _tc_pre(deg, zr):
    R, D = zr.shape

    def body(dg, zz, z_out, ns_out, nd_out):
        ns = lax.rsqrt(jnp.maximum(dg[0, :, 0:1], 1.0))
        nd = lax.rsqrt(jnp.maximum(dg[1, :, 0:1], 1.0))
        ns_out[...] = ns
        nd_out[...] = nd
        z = zz[...] * ns
        z_out[0] = z[:, :DH]
        z_out[1] = z[:, DH:]

    return pl.pallas_call(
        body,
        out_shape=(
            jax.ShapeDtypeStruct((NC, R, DH), jnp.float32),
            jax.ShapeDtypeStruct((R, 1), jnp.float32),
            jax.ShapeDtypeStruct((R, 1), jnp.float32),
        ),
    )(deg, zr)


def _tc_mid(p, ns, nd, b1, W2):
    _, R, _ = p.shape

    def body(p_ref, ns_ref, nd_ref, b_ref, w_ref, z_out):
        h = jnp.concatenate([p_ref[0], p_ref[1]], axis=1)
        h = jnp.maximum(h * nd_ref[...] + b_ref[...], 0.0)
        z = jnp.dot(h * ns_ref[...], w_ref[...],
                    preferred_element_type=jnp.float32)
        z_out[0] = z[:, :DH]
        z_out[1] = z[:, DH:]

    return pl.pallas_call(
        body,
        out_shape=jax.ShapeDtypeStruct((NC, R, DH), jnp.float32),
    )(p, ns, nd, b1, W2)


def _tc_post(p, nd, b2, n_rows):
    def body(p_ref, nd_ref, b_ref, out):
        h = jnp.concatenate([p_ref[0, :n_rows], p_ref[1, :n_rows]], axis=1)
        out[...] = h * nd_ref[:n_rows] + b_ref[...]

    return pl.pallas_call(
        body,
        out_shape=jax.ShapeDtypeStruct((n_rows, 2 * DH), jnp.float32),
    )(p, nd, b2)


def kernel(features, edge_index, W1, b1, W2, b2):
    N, D = features.shape
    E = edge_index.shape[1]

    R = pl.cdiv(N + 1, NS * CH) * (NS * CH)
    ept = pl.cdiv(E, NS * 16 * CH) * (16 * CH)
    nchunks = ept // CH
    pad = NS * ept - E

    src = edge_index[0]
    dst = edge_index[1]
    trash = (N + jnp.arange(pad, dtype=jnp.int32) % (R - N)).astype(jnp.int32)
    srcp = jnp.concatenate([src, trash]).reshape(NS, nchunks, CH)
    dstp = jnp.concatenate([dst, trash]).reshape(NS, nchunks, CH)
    edges = jnp.stack([srcp, dstp])

    feat = jnp.pad(features, ((0, R - N), (0, 0)))

    sc_deg = _make_sc_degrees(R, nchunks)
    sc_layer = _make_sc_layer(R, nchunks)

    deg = sc_deg(edges)
    zr = _tc_mm(feat, W1)
    z1, ns, nd = _tc_pre(deg, zr)
    p1 = sc_layer(z1, srcp, dstp)
    z2 = _tc_mid(p1, ns, nd, b1.reshape(1, D), W2)
    p2 = sc_layer(z2, srcp, dstp)
    return _tc_post(p2, nd, b2.reshape(1, D), N)
